# Initial kernel scaffold; baseline (speedup 1.0000x reference)
import jax
import jax.numpy as jnp
from jax.experimental import pallas as pl


def kernel(x, params, edge_index):
    raise NotImplementedError("write your pallas kernel here")


# speedup vs baseline: 1.0368x; 1.0368x over previous
import functools

import jax
import jax.numpy as jnp
import numpy as np
from jax.experimental import pallas as pl
from jax.experimental.pallas import tpu as pltpu

N = 10000
E = 160000
D = 256
HID = 256
BLK = 512
NP = 10240
NB = NP // BLK


def _rows(width):
    return pl.BlockSpec((BLK, width), lambda i: (i, 0))


def _full(shape):
    return pl.BlockSpec(shape, lambda i: tuple(0 for _ in shape))


def _stats_body(h_ref, o_ref):
    i = pl.program_id(0)

    @pl.when(i == 0)
    def _():
        o_ref[...] = jnp.zeros_like(o_ref)

    hb = h_ref[...]
    row = i * BLK + jax.lax.broadcasted_iota(jnp.int32, hb.shape, 0)
    hb = jnp.where(row < N, hb, 0.0)
    s = jnp.sum(hb, axis=0, keepdims=True)
    s2 = jnp.sum(hb * hb, axis=0, keepdims=True)
    o_ref[...] += jnp.concatenate([s, s2, jnp.zeros((6, HID), jnp.float32)], axis=0)


def _col_stats(h):
    return pl.pallas_call(
        _stats_body,
        grid=(NB,),
        in_specs=[_rows(HID)],
        out_specs=_full((8, HID)),
        out_shape=jax.ShapeDtypeStruct((8, HID), jnp.float32),
    )(h)


def _sage_body(agg_ref, h_ref, W_ref, b_ref, o_ref, *, act):
    xcat = jnp.concatenate([agg_ref[...], h_ref[...]], axis=1)
    o = jnp.dot(xcat, W_ref[...], preferred_element_type=jnp.float32)
    o = o + b_ref[...]
    o_ref[...] = jnp.maximum(o, 0.0) if act else o


def _sage_mm(agg, h, Wcat, b, act):
    return pl.pallas_call(
        functools.partial(_sage_body, act=act),
        grid=(NB,),
        in_specs=[_rows(HID), _rows(HID), _full((2 * HID, HID)), _full((1, HID))],
        out_specs=_rows(HID),
        out_shape=jax.ShapeDtypeStruct((NP, HID), jnp.float32),
    )(agg, h, Wcat, b)


def _gat_pre_body(h_ref, st_ref, gb_ref, W_ref, A_ref, hh_ref, s_ref):
    s = st_ref[0:1, :]
    s2 = st_ref[1:2, :]
    mu = s * (1.0 / N)
    var = s2 * (1.0 / N) - mu * mu
    rs = jax.lax.rsqrt(var + 1e-5)
    g = gb_ref[0:1, :]
    beta = gb_ref[1:2, :]
    xn = (h_ref[...] - mu) * (rs * g) + beta
    xn = jnp.maximum(xn, 0.0)
    hh = jnp.dot(xn, W_ref[...], preferred_element_type=jnp.float32)
    hh_ref[...] = hh
    s_ref[...] = jnp.dot(hh, A_ref[...], preferred_element_type=jnp.float32)


def _gat_pre(h, stats, gb, W, Acat):
    return pl.pallas_call(
        _gat_pre_body,
        grid=(NB,),
        in_specs=[_rows(HID), _full((8, HID)), _full((2, HID)),
                  _full((HID, 8 * HID)), _full((8 * HID, 128))],
        out_specs=[_rows(8 * HID), _rows(128)],
        out_shape=[jax.ShapeDtypeStruct((NP, 8 * HID), jnp.float32),
                   jax.ShapeDtypeStruct((NP, 128), jnp.float32)],
    )(h, stats, gb, W, Acat)


def _gat_post_body(agg_ref, d_ref, bias_ref, EX_ref, skW_ref, skb_ref, o_ref,
                   *, heads):
    rd = 1.0 / (d_ref[...][:, :heads] + 1e-16)
    rdx = jnp.dot(rd, EX_ref[...], preferred_element_type=jnp.float32)
    o = jnp.maximum(agg_ref[...] * rdx + bias_ref[...], 0.0)
    o2 = jnp.dot(o, skW_ref[...], preferred_element_type=jnp.float32)
    o_ref[...] = jnp.maximum(o2 + skb_ref[...], 0.0)


def _gat_post(agg, d, bias, EX, skW, skb, heads, width):
    return pl.pallas_call(
        functools.partial(_gat_post_body, heads=heads),
        grid=(NB,),
        in_specs=[_rows(width), _rows(128), _full((1, width)),
                  _full((heads, width)), _full((width, HID)), _full((1, HID))],
        out_specs=_rows(HID),
        out_shape=jax.ShapeDtypeStruct((NP, HID), jnp.float32),
    )(agg, d, bias, EX, skW, skb)


def _mm_body(x_ref, W_ref, b_ref, o_ref):
    o = jnp.dot(x_ref[...], W_ref[...], preferred_element_type=jnp.float32)
    o_ref[...] = o + b_ref[...]


def _mm(x, W, b, mout):
    k = x.shape[1]
    return pl.pallas_call(
        _mm_body,
        grid=(NB,),
        in_specs=[_rows(k), _full((k, mout)), _full((1, mout))],
        out_specs=_rows(mout),
        out_shape=jax.ShapeDtypeStruct((NP, mout), jnp.float32),
    )(x, W, b)


def _tr_post_body(agg_ref, d_ref, hs_ref, EX_ref, skW_ref, skb_ref, o_ref):
    rd = 1.0 / (d_ref[...][:, :4] + 1e-16)
    rdx = jnp.dot(rd, EX_ref[...], preferred_element_type=jnp.float32)
    o = jnp.maximum(agg_ref[...] * rdx + hs_ref[...], 0.0)
    o2 = jnp.dot(o, skW_ref[...], preferred_element_type=jnp.float32)
    o_ref[...] = jnp.maximum(o2 + skb_ref[...], 0.0)


def _tr_post(agg, d, hskip, EX, skW, skb):
    w = 4 * HID
    return pl.pallas_call(
        _tr_post_body,
        grid=(NB,),
        in_specs=[_rows(w), _rows(128), _rows(w),
                  _full((4, w)), _full((w, HID)), _full((1, HID))],
        out_specs=_rows(HID),
        out_shape=jax.ShapeDtypeStruct((NP, HID), jnp.float32),
    )(agg, d, hskip, EX, skW, skb)


def _seg_sum(vals, seg, num):
    return jax.ops.segment_sum(vals, seg, num_segments=num)


def _seg_softmax_unnorm(scores, seg, num):
    e = jnp.exp(scores)
    dsum = _seg_sum(e, seg, num)
    return e, dsum


def kernel(x, params, edge_index):
    p = params
    src = edge_index[0]
    dst = edge_index[1]

    xp = jnp.pad(x, ((0, NP - N), (0, 0)))

    ones = jnp.ones((E,), jnp.float32)
    deg = jnp.maximum(_seg_sum(ones, dst, N), 1.0)
    invdeg = (1.0 / deg)[:, None]

    def sage(h, i, act):
        hn = h[:N]
        agg = _seg_sum(hn[src], dst, N) * invdeg
        agg = jnp.pad(agg, ((0, NP - N), (0, 0)))
        Wcat = jnp.concatenate([p['sage%d_Wl' % i], p['sage%d_Wr' % i]], axis=0)
        return _sage_mm(agg, h, Wcat, p['sage%d_bl' % i][None, :], act)

    def gat(h, i):
        stats = _col_stats(h)
        gb = jnp.stack([p['bn%d_g' % i], p['bn%d_b' % i]])
        asrc = p['gat%d_asrc' % i]
        adst = p['gat%d_adst' % i]
        Acat = jnp.zeros((8 * HID, 128), jnp.float32)
        for hd in range(8):
            Acat = Acat.at[hd * HID:(hd + 1) * HID, hd].set(asrc[hd])
            Acat = Acat.at[hd * HID:(hd + 1) * HID, 8 + hd].set(adst[hd])
        hh, s_cat = _gat_pre(h, stats, gb, p['gat%d_W' % i], Acat)
        ssrc = s_cat[:N, :8]
        sdst = s_cat[:N, 8:16]
        a = ssrc[src] + sdst[dst]
        a = jnp.where(a > 0, a, 0.2 * a)
        e, dsum = _seg_softmax_unnorm(a, dst, N)
        hhn = hh[:N].reshape(N, 8, HID)
        agg = _seg_sum(e[:, :, None] * hhn[src], dst, N)
        agg = jnp.pad(agg.reshape(N, 8 * HID), ((0, NP - N), (0, 0)))
        dpad = jnp.pad(dsum, ((0, NP - N), (0, 120)))
        EX = jnp.repeat(jnp.eye(8, dtype=jnp.float32), HID, axis=1)
        return _gat_post(agg, dpad, p['gat%d_b' % i][None, :], EX,
                         p['skip%d_W' % i], p['skip%d_b' % i][None, :], 8, 8 * HID)

    def transf(h):
        c = HID
        Wcat = jnp.concatenate(
            [p['tr_Wq'] / 16.0, p['tr_Wk'], p['tr_Wv'], p['tr_Wskip']], axis=1)
        bcat = jnp.concatenate(
            [p['tr_bq'] / 16.0, p['tr_bk'], p['tr_bv'], p['tr_bskip']])[None, :]
        qkvs = _mm(h, Wcat, bcat, 16 * HID)
        q = qkvs[:N, 0:4 * HID].reshape(N, 4, c)
        k = qkvs[:N, 4 * HID:8 * HID].reshape(N, 4, c)
        v = qkvs[:N, 8 * HID:12 * HID]
        hskip = qkvs[:, 12 * HID:16 * HID]
        a = jnp.sum(q[dst] * k[src], -1)
        e, dsum = _seg_softmax_unnorm(a, dst, N)
        vn = v.reshape(N, 4, c)
        agg = _seg_sum(e[:, :, None] * vn[src], dst, N)
        agg = jnp.pad(agg.reshape(N, 4 * HID), ((0, NP - N), (0, 0)))
        dpad = jnp.pad(dsum, ((0, NP - N), (0, 124)))
        EX = jnp.repeat(jnp.eye(4, dtype=jnp.float32), HID, axis=1)
        return _tr_post(agg, dpad, hskip, EX, p['skip3_W'], p['skip3_b'][None, :])

    h = sage(xp, 1, True)
    h = gat(h, 1)
    h = sage(h, 2, True)
    h = gat(h, 2)
    h = sage(h, 3, True)
    h = transf(h)
    h = sage(h, 4, True)
    h = sage(h, 5, False)
    return h[:N]
